# Initial kernel scaffold; baseline (speedup 1.0000x reference)
import functools

import jax
import jax.numpy as jnp
from jax import lax
from jax.experimental import pallas as pl
from jax.experimental.pallas import tpu as pltpu
from jax.experimental.pallas import tpu_sc as plsc

_N = 10000
_E = 320000
_P = 320000
_HID = 64
_NC = 2
_NS = 16
_NW = _NC * _NS
_WIN = 80
_EPW = _E // _NW
_ROWS_PW = _EPW // _WIN


def _mesh():
    return plsc.VectorSubcoreMesh(core_axis_name="c", subcore_axis_name="s")


def _seg_body(compute_deg, table, src2d, dst2d, zfeat, zdeg, agg_out, deg_out,
              srcstage, dststage, rows, ones, accum, dega, sem):
    c = lax.axis_index("c")
    s = lax.axis_index("s")
    wid = s * _NC + c

    rpt = _N // _NS
    pltpu.sync_copy(zfeat.at[pl.ds(s * rpt, rpt)], accum.at[pl.ds(s * rpt, rpt)])
    if compute_deg:
        @pl.when(s < 5)
        def _():
            pltpu.sync_copy(zdeg.at[pl.ds(s * 2000, 2000)],
                            dega.at[pl.ds(s * 2000, 2000)])
        def _fill(i, _):
            ones[pl.ds(i * 16, 16)] = jnp.ones((16,), jnp.float32)
            return 0
        lax.fori_loop(0, _WIN // 16, _fill, 0)
    plsc.subcore_barrier()

    row0 = wid * _ROWS_PW
    nstage = 25

    def _window(w, _):
        r = row0 + w * nstage
        pltpu.sync_copy(src2d.at[pl.ds(r, nstage)], srcstage)
        pltpu.sync_copy(dst2d.at[pl.ds(r, nstage)], dststage)

        def _sub(k, _):
            pltpu.async_copy(table.at[srcstage.at[k]], rows, sem).wait()
            pltpu.sync_copy(rows, accum.at[dststage.at[k]], add=True)
            if compute_deg:
                pltpu.sync_copy(ones, dega.at[dststage.at[k]], add=True)
            return 0
        lax.fori_loop(0, nstage, _sub, 0)
        return 0
    lax.fori_loop(0, _ROWS_PW // nstage, _window, 0)

    plsc.subcore_barrier()
    pltpu.sync_copy(accum.at[pl.ds(s * rpt, rpt)],
                    agg_out.at[c, pl.ds(s * rpt, rpt)])
    if compute_deg:
        @pl.when(s < 5)
        def _():
            pltpu.sync_copy(dega.at[pl.ds(s * 2000, 2000)],
                            deg_out.at[c, pl.ds(s * 2000, 2000)])


def _make_seg_kernel(compute_deg):
    out_type = [jax.ShapeDtypeStruct((_NC, _N, _HID), jnp.float32),
                jax.ShapeDtypeStruct((_NC, _N), jnp.float32)]
    scratch = [
        pltpu.VMEM((25, _WIN), jnp.int32),
        pltpu.VMEM((25, _WIN), jnp.int32),
        pltpu.VMEM((_WIN, _HID), jnp.float32),
        pltpu.VMEM((_WIN,), jnp.float32),
        pltpu.VMEM_SHARED((_N, _HID), jnp.float32),
        pltpu.VMEM_SHARED((_N,), jnp.float32),
        pltpu.SemaphoreType.DMA,
    ]
    return pl.kernel(functools.partial(_seg_body, compute_deg),
                     out_type=out_type, mesh=_mesh(), scratch_types=scratch)


def _pair_body(tu, tv, u2d, v2d, t2d, w3t_h, w4_h, b4_h, y2d,
               ustage, vstage, tstage, bufu, bufv, ywin, m, w3c, w4c, b4c, sem):
    c = lax.axis_index("c")
    s = lax.axis_index("s")
    wid = s * _NC + c

    pltpu.sync_copy(w3t_h, w3c)
    pltpu.sync_copy(w4_h, w4c)
    pltpu.sync_copy(b4_h, b4c)

    w3tk = [w3c[pl.ds(16 * k, 16)] for k in range(4)]
    w4k = [w4c[pl.ds(16 * k, 16)] for k in range(4)]
    b4v = b4c[...]
    iota = lax.iota(jnp.int32, 16)

    row0 = wid * _ROWS_PW
    nstage = 25

    def _window(w, _):
        r = row0 + w * nstage
        pltpu.sync_copy(u2d.at[pl.ds(r, nstage)], ustage)
        pltpu.sync_copy(v2d.at[pl.ds(r, nstage)], vstage)
        pltpu.sync_copy(t2d.at[pl.ds(r, nstage)], tstage)

        def _sub(k, _):
            pltpu.async_copy(tu.at[ustage.at[k]], bufu, sem).wait()
            pltpu.async_copy(tv.at[vstage.at[k]], bufv, sem).wait()

            def _group(g, _):
                for j in range(16):
                    p = g * 16 + j
                    tsp = plsc.load_gather(
                        tstage, [jnp.zeros((16,), jnp.int32) + k,
                                 jnp.zeros((16,), jnp.int32) + p])
                    acc = None
                    for q in range(4):
                        zq = (bufu[p, pl.ds(16 * q, 16)]
                              + bufv[p, pl.ds(16 * q, 16)]
                              + tsp * w3tk[q])
                        rq = jnp.maximum(zq, 0.0) * w4k[q]
                        acc = rq if acc is None else acc + rq
                    m[pl.ds(j * 16, 16)] = acc
                ysum = b4v
                for cc in range(16):
                    ysum = ysum + plsc.load_gather(m, [iota * 16 + cc])
                ywin[k, pl.ds(g * 16, 16)] = ysum
                return 0
            lax.fori_loop(0, _WIN // 16, _group, 0)
            return 0
        lax.fori_loop(0, nstage, _sub, 0)
        pltpu.sync_copy(ywin, y2d.at[pl.ds(r, nstage)])
        return 0
    lax.fori_loop(0, _ROWS_PW // nstage, _window, 0)


def _make_pair_kernel():
    scratch = [
        pltpu.VMEM((25, _WIN), jnp.int32),
        pltpu.VMEM((25, _WIN), jnp.int32),
        pltpu.VMEM((25, _WIN), jnp.float32),
        pltpu.VMEM((_WIN, _HID), jnp.float32),
        pltpu.VMEM((_WIN, _HID), jnp.float32),
        pltpu.VMEM((25, _WIN), jnp.float32),
        pltpu.VMEM((256,), jnp.float32),
        pltpu.VMEM((_HID,), jnp.float32),
        pltpu.VMEM((_HID,), jnp.float32),
        pltpu.VMEM((16,), jnp.float32),
        pltpu.SemaphoreType.DMA,
    ]
    out_type = jax.ShapeDtypeStruct((_P // _WIN, _WIN), jnp.float32)
    return pl.kernel(_pair_body, out_type=out_type, mesh=_mesh(),
                     scratch_types=scratch)


def _tc1_body(x_ref, w_ref, xl_ref, xr_ref):
    out = jnp.dot(x_ref[...], w_ref[...], preferred_element_type=jnp.float32)
    xl_ref[...] = out[:, :_HID]
    xr_ref[...] = out[:, _HID:]


def _tc2_body(aggp_ref, degp_ref, other_ref, b_ref, w_ref, hl_ref, hr_ref):
    deg = degp_ref[0] + degp_ref[1]
    rdeg = 1.0 / jnp.maximum(deg, 1.0)
    h = jnp.maximum((aggp_ref[0] + aggp_ref[1]) * rdeg + b_ref[...]
                    + other_ref[...], 0.0)
    out = jnp.dot(h, w_ref[...], preferred_element_type=jnp.float32)
    hl_ref[...] = out[:, :_HID]
    hr_ref[...] = out[:, _HID:]


def _tc3_body(aggp_ref, degp_ref, other_ref, b_ref, w_ref, bias3_ref,
              tu_ref, tv_ref):
    deg = degp_ref[0] + degp_ref[1]
    rdeg = 1.0 / jnp.maximum(deg, 1.0)
    h = jnp.maximum((aggp_ref[0] + aggp_ref[1]) * rdeg + b_ref[...]
                    + other_ref[...], 0.0)
    out = (jnp.dot(h, w_ref[...], preferred_element_type=jnp.float32)
           + bias3_ref[...])
    tu_ref[...] = out[:, :_HID]
    tv_ref[...] = out[:, _HID:]


def kernel(x, edge_index, uv_pairs, t_feat, W1_l, b1_l, W1_r, W2_l, b2_l,
           W2_r, W3, b3, W4, b4):
    f32 = jnp.float32
    src2d = edge_index[0].reshape(_E // _WIN, _WIN)
    dst2d = edge_index[1].reshape(_E // _WIN, _WIN)
    u2d = uv_pairs[:, 0].reshape(_P // _WIN, _WIN)
    v2d = uv_pairs[:, 1].reshape(_P // _WIN, _WIN)
    t2d = t_feat.reshape(_P // _WIN, _WIN)

    wcat1 = jnp.concatenate([W1_l.T, W1_r.T], axis=1)
    wcat2 = jnp.concatenate([W2_l.T, W2_r.T], axis=1)
    wcat3 = jnp.concatenate([W3[:, :_HID].T, W3[:, _HID:2 * _HID].T], axis=1)
    bias3 = jnp.concatenate([jnp.zeros((_HID,), f32), b3]).reshape(1, 2 * _HID)
    w3t = W3[:, 2 * _HID]
    w4 = W4[0]
    b4s = jnp.broadcast_to(b4, (16,))
    zfeat = jnp.zeros((_N, _HID), f32)
    zdeg = jnp.zeros((_N,), f32)

    xl, xr = pl.pallas_call(
        _tc1_body,
        out_shape=[jax.ShapeDtypeStruct((_N, _HID), f32)] * 2,
    )(x, wcat1)

    agg1p, degp = _make_seg_kernel(True)(xl, src2d, dst2d, zfeat, zdeg)
    degp3 = degp.reshape(_NC, _N, 1)

    hl, hr = pl.pallas_call(
        _tc2_body,
        out_shape=[jax.ShapeDtypeStruct((_N, _HID), f32)] * 2,
    )(agg1p, degp3, xr, b1_l.reshape(1, _HID), wcat2)

    agg2p, _unused = _make_seg_kernel(False)(hl, src2d, dst2d, zfeat, zdeg)

    tu, tv = pl.pallas_call(
        _tc3_body,
        out_shape=[jax.ShapeDtypeStruct((_N, _HID), f32)] * 2,
    )(agg2p, degp3, hr, b2_l.reshape(1, _HID), wcat3, bias3)

    y2d = _make_pair_kernel()(tu, tv, u2d, v2d, t2d, w3t, w4, b4s)
    return y2d.reshape(_P)


# speedup vs baseline: 7.2920x; 7.2920x over previous
import jax
import jax.numpy as jnp
from jax import lax
from jax.experimental import pallas as pl
from jax.experimental.pallas import tpu as pltpu
from jax.experimental.pallas import tpu_sc as plsc

_N = 10000
_NPAD = 10240
_E = 320000
_P = 320000
_HID = 64
_TW = 128
_NC = 2
_NS = 16
_NW = _NC * _NS
_WIN = 128
_EPAD = 327680
_NROWS = _EPAD // _WIN
_ROWS_PW = _NROWS // _NW
_NSTAGE = 16


def _mesh():
    return plsc.VectorSubcoreMesh(core_axis_name="c", subcore_axis_name="s")


def _seg_body(table, src2d, dst2d, zfeat, agg_out,
              srcstage, dststage, rows, accum, sem):
    c = lax.axis_index("c")
    s = lax.axis_index("s")
    wid = s * _NC + c

    rpt = _NPAD // _NS
    pltpu.sync_copy(zfeat.at[pl.ds(s * rpt, rpt)], accum.at[pl.ds(s * rpt, rpt)])
    plsc.subcore_barrier()

    row0 = wid * _ROWS_PW

    def _window(w, _):
        r = row0 + w * _NSTAGE
        pltpu.sync_copy(src2d.at[pl.ds(r, _NSTAGE)], srcstage)
        pltpu.sync_copy(dst2d.at[pl.ds(r, _NSTAGE)], dststage)

        def _sub(k, _):
            pltpu.async_copy(table.at[srcstage.at[k]], rows, sem).wait()
            pltpu.sync_copy(rows, accum.at[dststage.at[k]], add=True)
            return 0
        lax.fori_loop(0, _NSTAGE, _sub, 0)
        return 0
    lax.fori_loop(0, _ROWS_PW // _NSTAGE, _window, 0)

    plsc.subcore_barrier()
    pltpu.sync_copy(accum.at[pl.ds(s * rpt, rpt)],
                    agg_out.at[c, pl.ds(s * rpt, rpt)])


def _make_seg_kernel():
    out_type = jax.ShapeDtypeStruct((_NC, _NPAD, _TW), jnp.float32)
    scratch = [
        pltpu.VMEM((_NSTAGE, _WIN), jnp.int32),
        pltpu.VMEM((_NSTAGE, _WIN), jnp.int32),
        pltpu.VMEM((_WIN, _TW), jnp.float32),
        pltpu.VMEM_SHARED((_NPAD, _TW), jnp.float32),
        pltpu.SemaphoreType.DMA,
    ]
    return pl.kernel(_seg_body, out_type=out_type, mesh=_mesh(),
                     scratch_types=scratch)


def _pair_body(tuv, u2d, v2d, t2d, w3t_h, w4_h, b4_h, y2d,
               ustage, vstage, tstage, bufu, bufv, ywin,
               w3c, w4c, b4c, sem):
    c = lax.axis_index("c")
    s = lax.axis_index("s")
    wid = s * _NC + c

    pltpu.sync_copy(w3t_h, w3c)
    pltpu.sync_copy(w4_h, w4c)
    pltpu.sync_copy(b4_h, b4c)

    w3tk = [w3c[pl.ds(16 * k, 16)] for k in range(4)]
    w4k = [w4c[pl.ds(16 * k, 16)] for k in range(4)]
    b4v = b4c[...]
    iota = lax.iota(jnp.int32, 16)

    row0 = wid * _ROWS_PW

    def _window(w, _):
        r = row0 + w * _NSTAGE
        pltpu.sync_copy(u2d.at[pl.ds(r, _NSTAGE)], ustage)
        pltpu.sync_copy(v2d.at[pl.ds(r, _NSTAGE)], vstage)
        pltpu.sync_copy(t2d.at[pl.ds(r, _NSTAGE)], tstage)

        def _sub(k, _):
            pltpu.async_copy(tuv.at[ustage.at[k]], bufu, sem).wait()
            pltpu.async_copy(tuv.at[vstage.at[k]], bufv, sem).wait()

            def _group(g, _):
                ysel = jnp.zeros((16,), jnp.float32)
                tg = tstage[k, pl.ds(g * 16, 16)]
                for j in range(16):
                    p = g * 16 + j
                    tsp = jnp.take(tg, jnp.full((16,), j, jnp.int32))
                    acc = None
                    for q in range(4):
                        zq = (bufu[p, pl.ds(16 * q, 16)]
                              + bufv[p, pl.ds(_HID + 16 * q, 16)]
                              + tsp * w3tk[q])
                        rq = jnp.maximum(zq, 0.0) * w4k[q]
                        acc = rq if acc is None else acc + rq
                    for sh in (8, 4, 2, 1):
                        acc = acc + jnp.take(acc, (iota + sh) % 16)
                    ysel = jnp.where(iota == j, acc, ysel)
                ywin[k, pl.ds(g * 16, 16)] = ysel + b4v
                return 0
            lax.fori_loop(0, _WIN // 16, _group, 0)
            return 0
        lax.fori_loop(0, _NSTAGE, _sub, 0)
        pltpu.sync_copy(ywin, y2d.at[pl.ds(r, _NSTAGE)])
        return 0
    lax.fori_loop(0, _ROWS_PW // _NSTAGE, _window, 0)


def _make_pair_kernel():
    scratch = [
        pltpu.VMEM((_NSTAGE, _WIN), jnp.int32),
        pltpu.VMEM((_NSTAGE, _WIN), jnp.int32),
        pltpu.VMEM((_NSTAGE, _WIN), jnp.float32),
        pltpu.VMEM((_WIN, _TW), jnp.float32),
        pltpu.VMEM((_WIN, _TW), jnp.float32),
        pltpu.VMEM((_NSTAGE, _WIN), jnp.float32),
        pltpu.VMEM((_HID,), jnp.float32),
        pltpu.VMEM((_HID,), jnp.float32),
        pltpu.VMEM((16,), jnp.float32),
        pltpu.SemaphoreType.DMA,
    ]
    out_type = jax.ShapeDtypeStruct((_NROWS, _WIN), jnp.float32)
    return pl.kernel(_pair_body, out_type=out_type, mesh=_mesh(),
                     scratch_types=scratch)


def _tc1_body(x_ref, w_ref, t1_ref, xr_ref):
    out = jnp.dot(x_ref[...], w_ref[...], preferred_element_type=jnp.float32)
    n = x_ref.shape[0]
    t1_ref[...] = jnp.concatenate(
        [out[:, :_HID],
         jnp.ones((n, 1), jnp.float32),
         jnp.zeros((n, _HID - 1), jnp.float32)], axis=1)
    xr_ref[...] = out[:, _HID:]


def _tc2_body(aggp_ref, xr_ref, b_ref, w_ref, t2_ref, rdeg_ref):
    deg = aggp_ref[0][:_N, _HID:_HID + 1] + aggp_ref[1][:_N, _HID:_HID + 1]
    rdeg = 1.0 / jnp.maximum(deg, 1.0)
    agg = aggp_ref[0][:_N, :_HID] + aggp_ref[1][:_N, :_HID]
    h = jnp.maximum(agg * rdeg + b_ref[...] + xr_ref[...], 0.0)
    t2_ref[...] = jnp.dot(h, w_ref[...], preferred_element_type=jnp.float32)
    rdeg_ref[...] = rdeg


def _tc3_body(aggp_ref, rdeg_ref, t2_ref, b_ref, w_ref, bias3_ref, tuv_ref):
    agg = aggp_ref[0][:_N, :_HID] + aggp_ref[1][:_N, :_HID]
    h = jnp.maximum(agg * rdeg_ref[...] + b_ref[...] + t2_ref[...][:, _HID:],
                    0.0)
    tuv_ref[...] = (jnp.dot(h, w_ref[...], preferred_element_type=jnp.float32)
                    + bias3_ref[...])


def kernel(x, edge_index, uv_pairs, t_feat, W1_l, b1_l, W1_r, W2_l, b2_l,
           W2_r, W3, b3, W4, b4):
    f32 = jnp.float32
    npad = _EPAD - _E
    padi = jnp.arange(npad, dtype=jnp.int32)
    src2d = jnp.concatenate([edge_index[0], padi % _N]).reshape(_NROWS, _WIN)
    dst2d = jnp.concatenate(
        [edge_index[1], _N + padi % (_NPAD - _N)]).reshape(_NROWS, _WIN)
    u2d = jnp.concatenate([uv_pairs[:, 0], padi % _N]).reshape(_NROWS, _WIN)
    v2d = jnp.concatenate([uv_pairs[:, 1], padi % _N]).reshape(_NROWS, _WIN)
    t2d = jnp.concatenate([t_feat[:, 0], jnp.zeros((npad,), f32)]
                          ).reshape(_NROWS, _WIN)

    wcat1 = jnp.concatenate([W1_l.T, W1_r.T], axis=1)
    wcat2 = jnp.concatenate([W2_l.T, W2_r.T], axis=1)
    wcat3 = jnp.concatenate([W3[:, :_HID].T, W3[:, _HID:2 * _HID].T], axis=1)
    bias3 = jnp.concatenate([jnp.zeros((_HID,), f32), b3]).reshape(1, 2 * _HID)
    w3t = W3[:, 2 * _HID]
    w4 = W4[0]
    b4s = jnp.broadcast_to(b4, (16,))
    zfeat = jnp.zeros((_NPAD, _TW), f32)

    table1, xr = pl.pallas_call(
        _tc1_body,
        out_shape=[jax.ShapeDtypeStruct((_N, _TW), f32),
                   jax.ShapeDtypeStruct((_N, _HID), f32)],
    )(x, wcat1)

    seg = _make_seg_kernel()
    agg1p = seg(table1, src2d, dst2d, zfeat)

    table2, rdeg = pl.pallas_call(
        _tc2_body,
        out_shape=[jax.ShapeDtypeStruct((_N, _TW), f32),
                   jax.ShapeDtypeStruct((_N, 1), f32)],
    )(agg1p, xr, b1_l.reshape(1, _HID), wcat2)

    agg2p = seg(table2, src2d, dst2d, zfeat)

    tuv = pl.pallas_call(
        _tc3_body,
        out_shape=jax.ShapeDtypeStruct((_N, _TW), f32),
    )(agg2p, rdeg, table2, b2_l.reshape(1, _HID), wcat3, bias3)

    y2d = _make_pair_kernel()(tuv, u2d, v2d, t2d, w3t, w4, b4s)
    return y2d.reshape(_EPAD)[:_P]
